# baseline (speedup 1.0000x reference)
import jax
import jax.numpy as jnp
from jax.experimental import pallas as pl

DIM = 256
BLK_ROWS = 2048


def _body(ys_ref, m_ref, w_ref, b_ref, mt_ref, y_ref, yq_ref):
    ys = ys_ref[...]
    m = m_ref[...]
    w = w_ref[...]
    b = b_ref[...]
    y_ref[...] = (ys * w + b) * m
    yq_ref[...] = jnp.broadcast_to(mt_ref[...], (BLK_ROWS, DIM))


def kernel(y_support, padding_obs_support, n_obs_query, W_y, b_y, mask_table):
    batch, n_sup = y_support.shape
    total = batch * n_sup
    ys2 = y_support.reshape(total, 1)
    m2 = jnp.where(padding_obs_support.reshape(total, 1), 0.0, 1.0).astype(jnp.float32)
    w2 = W_y.reshape(1, DIM)
    b2 = b_y.reshape(1, DIM)

    y, yq = pl.pallas_call(
        _body,
        grid=(total // BLK_ROWS,),
        in_specs=[
            pl.BlockSpec((BLK_ROWS, 1), lambda i: (i, 0)),
            pl.BlockSpec((BLK_ROWS, 1), lambda i: (i, 0)),
            pl.BlockSpec((1, DIM), lambda i: (0, 0)),
            pl.BlockSpec((1, DIM), lambda i: (0, 0)),
            pl.BlockSpec((1, DIM), lambda i: (0, 0)),
        ],
        out_specs=[
            pl.BlockSpec((BLK_ROWS, DIM), lambda i: (i, 0)),
            pl.BlockSpec((BLK_ROWS, DIM), lambda i: (i, 0)),
        ],
        out_shape=[
            jax.ShapeDtypeStruct((total, DIM), jnp.float32),
            jax.ShapeDtypeStruct((total, DIM), jnp.float32),
        ],
    )(ys2, m2, w2, b2, mask_table)

    return (
        y.reshape(batch, n_sup, DIM),
        yq.reshape(batch, n_sup, 1, DIM),
    )


# speedup vs baseline: 1.4815x; 1.4815x over previous
import functools

import jax
import jax.numpy as jnp
from jax import lax
from jax.experimental import pallas as pl
from jax.experimental.pallas import tpu as pltpu
from jax.experimental.pallas import tpu_sc as plsc

DIM = 256
BLK_MAJ = 8
BLK_R = 1024
BUF_ROWS = 256
LANES = 16


def _tc_body(ys_ref, m_ref, w_ref, b_ref, y_ref):
    ys = ys_ref[...]
    m = m_ref[...]
    w = w_ref[0, :]
    b = b_ref[0, :]
    y_ref[...] = (ys[:, :, None] * w[None, None, :] + b[None, None, :]) * m[:, :, None]


def _tc_y(ys2, m2, w2, b2):
    G, R = ys2.shape
    return pl.pallas_call(
        _tc_body,
        grid=(G // BLK_MAJ,),
        in_specs=[
            pl.BlockSpec((BLK_MAJ, R), lambda i: (i, 0)),
            pl.BlockSpec((BLK_MAJ, R), lambda i: (i, 0)),
            pl.BlockSpec((1, DIM), lambda i: (0, 0)),
            pl.BlockSpec((1, DIM), lambda i: (0, 0)),
        ],
        out_specs=pl.BlockSpec((BLK_MAJ, R, DIM), lambda i: (i, 0, 0)),
        out_shape=jax.ShapeDtypeStruct((G, R, DIM), jnp.float32),
    )(ys2, m2, w2, b2)


def _sc_fill(mask_table, total):
    info = plsc.get_sparse_core_info()
    nc, ns = info.num_cores, info.num_subcores
    nw = nc * ns
    rows_per_w = total // nw
    n_chunk = rows_per_w // BUF_ROWS
    mesh = plsc.VectorSubcoreMesh(core_axis_name="c", subcore_axis_name="s")

    @functools.partial(
        pl.kernel,
        out_type=jax.ShapeDtypeStruct((total, DIM), jnp.float32),
        mesh=mesh,
        scratch_types=[
            pltpu.VMEM((1, DIM), jnp.float32),
            pltpu.VMEM((BUF_ROWS, DIM), jnp.float32),
        ],
    )
    def yq_fill(mt_hbm, out_hbm, row_v, buf_v):
        wid = lax.axis_index("s") * nc + lax.axis_index("c")
        base = wid * rows_per_w
        pltpu.sync_copy(mt_hbm, row_v)
        vs = [row_v[0, pl.ds(LANES * d, LANES)] for d in range(DIM // LANES)]

        def fill_body(i, carry):
            for d in range(DIM // LANES):
                buf_v[i, pl.ds(LANES * d, LANES)] = vs[d]
            return carry

        lax.fori_loop(0, BUF_ROWS, fill_body, 0)

        def out_body(j, carry):
            pltpu.sync_copy(
                buf_v, out_hbm.at[pl.ds(base + j * BUF_ROWS, BUF_ROWS)]
            )
            return carry

        lax.fori_loop(0, n_chunk, out_body, 0)

    return yq_fill(mask_table)


def kernel(y_support, padding_obs_support, n_obs_query, W_y, b_y, mask_table):
    batch, n_sup = y_support.shape
    total = batch * n_sup
    R = BLK_R
    G = total // R
    ys2 = y_support.reshape(G, R)
    m2 = jnp.where(padding_obs_support.reshape(G, R), 0.0, 1.0).astype(jnp.float32)
    w2 = W_y.reshape(1, DIM)
    b2 = b_y.reshape(1, DIM)

    yq = _sc_fill(mask_table, total)
    y = _tc_y(ys2, m2, w2, b2)

    return (
        y.reshape(batch, n_sup, DIM),
        yq.reshape(batch, n_sup, 1, DIM),
    )


# speedup vs baseline: 1.4816x; 1.0000x over previous
import functools

import jax
import jax.numpy as jnp
from jax import lax
from jax.experimental import pallas as pl
from jax.experimental.pallas import tpu as pltpu
from jax.experimental.pallas import tpu_sc as plsc

DIM = 256
BLK_MAJ = 8
BLK_R = 1024
BUF_ROWS = 256
LANES = 16


def _tc_body(ys_ref, m_ref, w_ref, b_ref, y_ref):
    ys = ys_ref[...]
    m = m_ref[...]
    w = w_ref[0, :]
    b = b_ref[0, :]
    y_ref[...] = (ys[:, :, None] * w[None, None, :] + b[None, None, :]) * m[:, :, None]


def _tc_y(ys2, m2, w2, b2):
    G, R = ys2.shape
    return pl.pallas_call(
        _tc_body,
        grid=(G // BLK_MAJ,),
        in_specs=[
            pl.BlockSpec((BLK_MAJ, R), lambda i: (i, 0)),
            pl.BlockSpec((BLK_MAJ, R), lambda i: (i, 0)),
            pl.BlockSpec((1, DIM), lambda i: (0, 0)),
            pl.BlockSpec((1, DIM), lambda i: (0, 0)),
        ],
        out_specs=pl.BlockSpec((BLK_MAJ, R, DIM), lambda i: (i, 0, 0)),
        out_shape=jax.ShapeDtypeStruct((G, R, DIM), jnp.float32),
    )(ys2, m2, w2, b2)


def _sc_fill(mask_table, total):
    info = plsc.get_sparse_core_info()
    nc, ns = info.num_cores, info.num_subcores
    nw = nc * ns
    rows_per_w = total // nw
    n_chunk = rows_per_w // BUF_ROWS
    mesh = plsc.VectorSubcoreMesh(core_axis_name="c", subcore_axis_name="s")

    @functools.partial(
        pl.kernel,
        out_type=jax.ShapeDtypeStruct((total, DIM), jnp.float32),
        mesh=mesh,
        compiler_params=pltpu.CompilerParams(use_tc_tiling_on_sc=True),
        scratch_types=[
            pltpu.VMEM((1, DIM), jnp.float32),
            pltpu.VMEM((BUF_ROWS, DIM), jnp.float32),
        ],
    )
    def yq_fill(mt_hbm, out_hbm, row_v, buf_v):
        wid = lax.axis_index("s") * nc + lax.axis_index("c")
        base = wid * rows_per_w
        pltpu.sync_copy(mt_hbm, row_v)
        vs = [row_v[0, pl.ds(LANES * d, LANES)] for d in range(DIM // LANES)]

        def fill_body(i, carry):
            for d in range(DIM // LANES):
                buf_v[i, pl.ds(LANES * d, LANES)] = vs[d]
            return carry

        lax.fori_loop(0, BUF_ROWS, fill_body, 0)

        def out_body(j, carry):
            pltpu.sync_copy(
                buf_v, out_hbm.at[pl.ds(base + j * BUF_ROWS, BUF_ROWS)]
            )
            return carry

        lax.fori_loop(0, n_chunk, out_body, 0)

    return yq_fill(mask_table)


def kernel(y_support, padding_obs_support, n_obs_query, W_y, b_y, mask_table):
    batch, n_sup = y_support.shape
    total = batch * n_sup
    R = BLK_R
    G = total // R
    ys2 = y_support.reshape(G, R)
    m2 = jnp.where(padding_obs_support.reshape(G, R), 0.0, 1.0).astype(jnp.float32)
    w2 = W_y.reshape(1, DIM)
    b2 = b_y.reshape(1, DIM)

    yq = _sc_fill(mask_table, total)
    y = _tc_y(ys2, m2, w2, b2)

    return (
        y.reshape(batch, n_sup, DIM),
        yq.reshape(batch, n_sup, 1, DIM),
    )


# speedup vs baseline: 2.9542x; 1.9940x over previous
import functools

import jax
import jax.numpy as jnp
from jax import lax
from jax.experimental import pallas as pl
from jax.experimental.pallas import tpu as pltpu
from jax.experimental.pallas import tpu_sc as plsc

DIM = 256
BLK_MAJ = 8
BLK_R = 1024
BUF_ROWS = 256
LANES = 16


def _tc_body(ys_ref, m_ref, w_ref, b_ref, y_ref):
    ys = ys_ref[...]
    m = m_ref[...]
    w = w_ref[0, :]
    b = b_ref[0, :]
    y_ref[...] = (ys[:, :, None] * w[None, None, :] + b[None, None, :]) * m[:, :, None]


def _tc_y(ys2, m2, w2, b2):
    G, R = ys2.shape
    return pl.pallas_call(
        _tc_body,
        grid=(G // BLK_MAJ,),
        in_specs=[
            pl.BlockSpec((BLK_MAJ, R), lambda i: (i, 0)),
            pl.BlockSpec((BLK_MAJ, R), lambda i: (i, 0)),
            pl.BlockSpec((1, DIM), lambda i: (0, 0)),
            pl.BlockSpec((1, DIM), lambda i: (0, 0)),
        ],
        out_specs=pl.BlockSpec((BLK_MAJ, R, DIM), lambda i: (i, 0, 0)),
        out_shape=jax.ShapeDtypeStruct((G, R, DIM), jnp.float32),
    )(ys2, m2, w2, b2)


def _sc_fill(mask_table, batch, n_query):
    info = plsc.get_sparse_core_info()
    nc, ns = info.num_cores, info.num_subcores
    nw = nc * ns
    total = batch * n_query
    rows_per_w = total // nw
    n_chunk = rows_per_w // BUF_ROWS
    w_per_b = n_query // rows_per_w
    mesh = plsc.VectorSubcoreMesh(core_axis_name="c", subcore_axis_name="s")

    @functools.partial(
        pl.kernel,
        out_type=jax.ShapeDtypeStruct((batch, n_query, 1, DIM), jnp.float32),
        mesh=mesh,
        scratch_types=[
            pltpu.VMEM((1, DIM), jnp.float32),
            pltpu.VMEM((BUF_ROWS, DIM), jnp.float32),
        ],
    )
    def yq_fill(mt_hbm, out_hbm, row_v, buf_v):
        wid = lax.axis_index("s") * nc + lax.axis_index("c")
        b = wid // w_per_b
        n0 = (wid % w_per_b) * rows_per_w
        pltpu.sync_copy(mt_hbm, row_v)
        vs = [row_v[0, pl.ds(LANES * d, LANES)] for d in range(DIM // LANES)]

        def fill_body(i, carry):
            for d in range(DIM // LANES):
                buf_v[i, pl.ds(LANES * d, LANES)] = vs[d]
            return carry

        lax.fori_loop(0, BUF_ROWS, fill_body, 0)

        def out_body(j, carry):
            pltpu.sync_copy(
                buf_v,
                out_hbm.at[b, pl.ds(n0 + j * BUF_ROWS, BUF_ROWS), 0],
            )
            return carry

        lax.fori_loop(0, n_chunk, out_body, 0)

    return yq_fill(mask_table)


def kernel(y_support, padding_obs_support, n_obs_query, W_y, b_y, mask_table):
    batch, n_sup = y_support.shape
    total = batch * n_sup
    R = BLK_R
    G = total // R
    ys2 = y_support.reshape(G, R)
    m2 = jnp.where(padding_obs_support.reshape(G, R), 0.0, 1.0).astype(jnp.float32)
    w2 = W_y.reshape(1, DIM)
    b2 = b_y.reshape(1, DIM)

    yq = _sc_fill(mask_table, batch, n_sup)
    y = _tc_y(ys2, m2, w2, b2)

    return (
        y.reshape(batch, n_sup, DIM),
        yq,
    )
